# baseline (speedup 1.0000x reference)
import jax
import jax.numpy as jnp
from jax.experimental import pallas as pl
from jax.experimental.pallas import tpu as pltpu

_PACK = 8
_BLOCK = 2000


def _mpn_block_kernel(x1_ref, x2_ref, x3_ref, w1_ref, w2_ref, w3_ref, b_ref,
                      out_ref):
    acc = jnp.dot(x1_ref[...], w1_ref[...], preferred_element_type=jnp.float32)
    acc = acc + jnp.dot(x2_ref[...], w2_ref[...],
                        preferred_element_type=jnp.float32)
    acc = acc + jnp.dot(x3_ref[...], w3_ref[...],
                        preferred_element_type=jnp.float32)
    acc = acc + b_ref[...]
    out_ref[...] = jnp.maximum(acc, 0.0)


def kernel(f_src, f, sum_msg, w1, w2, w3, b):
    e, d_ndata = f_src.shape
    d_edata = f.shape[1]
    d_msg = sum_msg.shape[1]

    ep = e // _PACK
    eye = jnp.eye(_PACK, dtype=jnp.float32)
    x1 = f_src.reshape(ep, _PACK * d_ndata)
    x2 = f.reshape(ep, _PACK * d_edata)
    x3 = sum_msg.reshape(ep, _PACK * d_msg)
    w1b = jnp.kron(eye, w1)
    w2b = jnp.kron(eye, w2)
    w3b = jnp.kron(eye, w3)
    bt = jnp.tile(b, (1, _PACK))

    block = _BLOCK if ep % _BLOCK == 0 else ep
    grid = ep // block

    out = pl.pallas_call(
        _mpn_block_kernel,
        grid=(grid,),
        in_specs=[
            pl.BlockSpec((block, _PACK * d_ndata), lambda i: (i, 0)),
            pl.BlockSpec((block, _PACK * d_edata), lambda i: (i, 0)),
            pl.BlockSpec((block, _PACK * d_msg), lambda i: (i, 0)),
            pl.BlockSpec((_PACK * d_ndata, _PACK * d_msg), lambda i: (0, 0)),
            pl.BlockSpec((_PACK * d_edata, _PACK * d_msg), lambda i: (0, 0)),
            pl.BlockSpec((_PACK * d_msg, _PACK * d_msg), lambda i: (0, 0)),
            pl.BlockSpec((1, _PACK * d_msg), lambda i: (0, 0)),
        ],
        out_specs=pl.BlockSpec((block, _PACK * d_msg), lambda i: (i, 0)),
        out_shape=jax.ShapeDtypeStruct((ep, _PACK * d_msg), jnp.float32),
        compiler_params=pltpu.CompilerParams(
            dimension_semantics=("parallel",)),
    )(x1, x2, x3, w1b, w2b, w3b, bt)
    return out.reshape(e, d_msg)
